# baseline (speedup 1.0000x reference)
import jax
import jax.numpy as jnp
from jax import lax
from jax.experimental import pallas as pl
from jax.experimental.pallas import tpu as pltpu
from jax.experimental.pallas import tpu_sc as plsc

_L = 16
_QPW = 512


def _sc_nn(qt_hbm, qtb_hbm, ptb_hbm, pp_hbm, best_hbm, bidx_hbm,
           q_v, qb_v, p_v, pp_v, best_v, bidx_v):
    c = lax.axis_index("c")
    s = lax.axis_index("s")
    wid = s * 2 + c
    b = wid // 4
    quarter = wid % 4
    P = ptb_hbm.shape[2]
    n_vregs = P // _L

    pltpu.sync_copy(qt_hbm.at[b, :, pl.ds(quarter * _QPW, _QPW)], q_v)
    pltpu.sync_copy(qtb_hbm.at[b, :, pl.ds(quarter * _QPW, _QPW)], qb_v)
    pltpu.sync_copy(ptb_hbm.at[b], p_v)
    pltpu.sync_copy(pp_hbm.at[b], pp_v)

    def group_body(g, _):
        gq0 = q_v[0, pl.ds(g * _L, _L)]
        gq1 = q_v[1, pl.ds(g * _L, _L)]
        gq2 = q_v[2, pl.ds(g * _L, _L)]
        qqv = (gq0 * gq0 + gq1 * gq1) + gq2 * gq2
        gb0 = qb_v[0, pl.ds(g * _L, _L)]
        gb1 = qb_v[1, pl.ds(g * _L, _L)]
        gb2 = qb_v[2, pl.ds(g * _L, _L)]

        for j in range(_L):
            b0 = gb0[j]
            b1 = gb1[j]
            b2 = gb2[j]
            qq = qqv[j]

            def point_body(i, carry):
                best, bidx = carry
                px = p_v[0, pl.ds(i * _L, _L)]
                py = p_v[1, pl.ds(i * _L, _L)]
                pz = p_v[2, pl.ds(i * _L, _L)]
                pp = pp_v[0, pl.ds(i * _L, _L)]
                t = (b0 * px + b1 * py) + b2 * pz
                key = (qq - (t + t)) + pp
                lt = key < best
                return (jnp.where(lt, key, best),
                        jnp.where(lt, jnp.full((_L,), i, jnp.int32), bidx))

            best0 = jnp.full((_L,), jnp.inf, jnp.float32)
            bidx0 = jnp.zeros((_L,), jnp.int32)
            best, bidx = lax.fori_loop(0, n_vregs, point_body,
                                       (best0, bidx0), unroll=4)
            best_v[pl.ds((g * _L + j) * _L, _L)] = best
            bidx_v[pl.ds((g * _L + j) * _L, _L)] = bidx
        return 0

    lax.fori_loop(0, _QPW // _L, group_body, 0)
    pltpu.sync_copy(best_v, best_hbm.at[b, pl.ds(quarter * _QPW * _L, _QPW * _L)])
    pltpu.sync_copy(bidx_v, bidx_hbm.at[b, pl.ds(quarter * _QPW * _L, _QPW * _L)])


def _finish_body(best_ref, bidx_ref, out_ref):
    best = best_ref[0]
    bidx = bidx_ref[0]
    P = 8192
    m = jnp.min(best, axis=0, keepdims=True)
    lane = lax.broadcasted_iota(jnp.int32, (16, best.shape[1]), 0)
    full = bidx * _L + lane
    idx = jnp.min(jnp.where(best == m, full, P), axis=0)
    out_ref[0, 0, :] = idx


def kernel(queries, points):
    B, Q, _ = queries.shape
    P = points.shape[1]
    qt = queries.transpose(0, 2, 1)
    qtb = lax.reduce_precision(qt, 8, 7)
    ptb = lax.reduce_precision(points.transpose(0, 2, 1), 8, 7)
    pp = jnp.sum(points * points, axis=-1)[:, None, :]

    mesh = plsc.VectorSubcoreMesh(core_axis_name="c", subcore_axis_name="s")
    f = pl.kernel(
        _sc_nn,
        mesh=mesh,
        out_type=(
            jax.ShapeDtypeStruct((B, Q * _L), jnp.float32),
            jax.ShapeDtypeStruct((B, Q * _L), jnp.int32),
        ),
        scratch_types=[
            pltpu.VMEM((3, _QPW), jnp.float32),
            pltpu.VMEM((3, _QPW), jnp.float32),
            pltpu.VMEM((3, P), jnp.float32),
            pltpu.VMEM((1, P), jnp.float32),
            pltpu.VMEM((_QPW * _L,), jnp.float32),
            pltpu.VMEM((_QPW * _L,), jnp.int32),
        ],
    )
    best, bidx = f(qt, qtb, ptb, pp)
    best_t = best.reshape(B, Q, _L).transpose(0, 2, 1)
    bidx_t = bidx.reshape(B, Q, _L).transpose(0, 2, 1)
    qb = 512
    nq = Q // qb
    out = pl.pallas_call(
        _finish_body,
        grid=(B, nq),
        in_specs=[
            pl.BlockSpec((1, _L, qb), lambda b, i: (b, 0, i)),
            pl.BlockSpec((1, _L, qb), lambda b, i: (b, 0, i)),
        ],
        out_specs=pl.BlockSpec((1, 1, qb), lambda b, i: (b * nq + i, 0, 0)),
        out_shape=jax.ShapeDtypeStruct((B * nq, 1, qb), jnp.int32),
    )(best_t, bidx_t)
    return out.reshape(B, Q).astype(jnp.int64)


# speedup vs baseline: 4.1753x; 4.1753x over previous
import jax
import jax.numpy as jnp
from jax import lax
from jax.experimental import pallas as pl

_QB = 512
_C = 512


def _nn_body(q_ref, pt_ref, out_ref):
    P = pt_ref.shape[2]
    n_chunks = P // _C
    q = q_ref[0]
    qbm2 = q.astype(jnp.bfloat16) * jnp.bfloat16(-2.0)
    q0 = q[:, 0:1]
    q1 = q[:, 1:2]
    q2 = q[:, 2:3]
    qq = q0 * q0 + q1 * q1 + q2 * q2

    def chunk_key(c):
        pt = pt_ref[0, :, pl.ds(c * _C, _C)]
        qp2 = lax.dot_general(
            qbm2, pt.astype(jnp.bfloat16),
            dimension_numbers=(((1,), (0,)), ((), ())),
            preferred_element_type=jnp.float32)
        px = pt[0:1, :]
        py = pt[1:2, :]
        pz = pt[2:3, :]
        pp = px * px + py * py + pz * pz
        return (qq + qp2) + pp

    nodes = [(chunk_key(c), c) for c in range(n_chunks)]
    while len(nodes) > 1:
        nxt = []
        for j in range(0, len(nodes), 2):
            (ka, ia), (kb, ib) = nodes[j], nodes[j + 1]
            lt = kb < ka
            k = jnp.where(lt, kb, ka)
            if isinstance(ia, int) and isinstance(ib, int):
                i = jnp.where(lt, ib, ia)
            else:
                i = jnp.where(lt, ib, ia)
            nxt.append((k, i))
        nodes = nxt
    best, bidx = nodes[0]

    m = jnp.min(best, axis=1, keepdims=True)
    lane = lax.broadcasted_iota(jnp.int32, (_QB, _C), 1)
    full = bidx * _C + lane
    idx = jnp.min(jnp.where(best == m, full, P), axis=1)
    out_ref[0, 0, :] = idx


def kernel(queries, points):
    B, Q, _ = queries.shape
    P = points.shape[1]
    pt = points.transpose(0, 2, 1)
    nq = Q // _QB
    out = pl.pallas_call(
        _nn_body,
        grid=(B, nq),
        in_specs=[
            pl.BlockSpec((1, _QB, 3), lambda b, i: (b, i, 0)),
            pl.BlockSpec((1, 3, P), lambda b, i: (b, 0, 0)),
        ],
        out_specs=pl.BlockSpec((1, 1, _QB), lambda b, i: (b * nq + i, 0, 0)),
        out_shape=jax.ShapeDtypeStruct((B * nq, 1, _QB), jnp.int32),
    )(queries, pt)
    return out.reshape(B, Q).astype(jnp.int64)
